# baseline (speedup 1.0000x reference)
import functools

import jax
import jax.numpy as jnp
from jax.experimental import pallas as pl
from jax.experimental.pallas import tpu as pltpu

R = 128
N = 100000
BC = 2048
NB = (N + BC - 1) // BC
_I32_MAX = jnp.iinfo(jnp.int32).max


def _argmax_kernel(x_ref, idx_ref, vmax_ref, vidx_ref):
    j = pl.program_id(0)
    base = j * BC
    x = x_ref[...]
    cols = jax.lax.broadcasted_iota(jnp.int32, x.shape, 1) + base
    xm = jnp.where(cols < N, x, -jnp.inf)
    m = jnp.max(xm, axis=-1, keepdims=True)
    li = jnp.min(jnp.where(xm == m, cols, _I32_MAX),
                 axis=-1, keepdims=True)

    @pl.when(j == 0)
    def _():
        vmax_ref[...] = m
        vidx_ref[...] = li

    @pl.when(j > 0)
    def _():
        better = m > vmax_ref[...]
        vmax_ref[...] = jnp.where(better, m, vmax_ref[...])
        vidx_ref[...] = jnp.where(better, li, vidx_ref[...])

    @pl.when(j == NB - 1)
    def _():
        idx_ref[...] = vidx_ref[...]


def _onehot_kernel(idx_ref, o_ref):
    j = pl.program_id(0)
    base = j * BC
    cols = jax.lax.broadcasted_iota(jnp.int32, o_ref.shape, 1) + base
    o_ref[...] = (cols == idx_ref[...]).astype(jnp.float32)


@functools.partial(jax.jit, static_argnames=("interpret",))
def kernel(logits, interpret=False):
    idx = pl.pallas_call(
        _argmax_kernel,
        grid=(NB,),
        in_specs=[pl.BlockSpec((R, BC), lambda j: (0, j))],
        out_specs=pl.BlockSpec((R, 1), lambda j: (0, 0)),
        out_shape=jax.ShapeDtypeStruct((R, 1), jnp.int32),
        scratch_shapes=[
            pltpu.VMEM((R, 1), jnp.float32),
            pltpu.VMEM((R, 1), jnp.int32),
        ],
        interpret=interpret,
    )(logits)

    out = pl.pallas_call(
        _onehot_kernel,
        grid=(NB,),
        in_specs=[pl.BlockSpec((R, 1), lambda j: (0, 0))],
        out_specs=pl.BlockSpec((R, BC), lambda j: (0, j)),
        out_shape=jax.ShapeDtypeStruct((R, N), jnp.float32),
        interpret=interpret,
    )(idx)
    return out


# speedup vs baseline: 1.2223x; 1.2223x over previous
import functools

import jax
import jax.numpy as jnp
from jax.experimental import pallas as pl
from jax.experimental.pallas import tpu as pltpu

R = 128
N = 100000
BC = 8192
NB = (N + BC - 1) // BC
_I32_MAX = jnp.iinfo(jnp.int32).max


def _argmax_kernel(x_ref, idx_ref, acc_ref, blk_ref):
    j = pl.program_id(0)
    x = x_ref[...]

    @pl.when(j == NB - 1)
    def _():
        lane = jax.lax.broadcasted_iota(jnp.int32, x.shape, 1)
        xm = jnp.where(lane < N - (NB - 1) * BC, x, -jnp.inf)
        better = xm > acc_ref[...]
        acc = jnp.where(better, xm, acc_ref[...])
        blk = jnp.where(better, j, blk_ref[...])
        m = jnp.max(acc, axis=-1, keepdims=True)
        cand = jnp.where(acc == m, blk * BC + lane, _I32_MAX)
        idx_ref[...] = jnp.min(cand, axis=-1, keepdims=True)

    @pl.when(j == 0)
    def _():
        acc_ref[...] = x
        blk_ref[...] = jnp.zeros_like(blk_ref)

    @pl.when(jnp.logical_and(j > 0, j < NB - 1))
    def _():
        better = x > acc_ref[...]
        acc_ref[...] = jnp.where(better, x, acc_ref[...])
        blk_ref[...] = jnp.where(better, j, blk_ref[...])


def _onehot_kernel(idx_ref, o_ref):
    j = pl.program_id(0)
    lt = idx_ref[...] - j * BC
    lane = jax.lax.broadcasted_iota(jnp.int32, o_ref.shape, 1)
    o_ref[...] = (lane == lt).astype(jnp.float32)


@functools.partial(jax.jit, static_argnames=("interpret",))
def kernel(logits, interpret=False):
    idx = pl.pallas_call(
        _argmax_kernel,
        grid=(NB,),
        in_specs=[pl.BlockSpec((R, BC), lambda j: (0, j))],
        out_specs=pl.BlockSpec((R, 1), lambda j: (0, 0)),
        out_shape=jax.ShapeDtypeStruct((R, 1), jnp.int32),
        scratch_shapes=[
            pltpu.VMEM((R, BC), jnp.float32),
            pltpu.VMEM((R, BC), jnp.int32),
        ],
        interpret=interpret,
    )(logits)

    out = pl.pallas_call(
        _onehot_kernel,
        grid=(NB,),
        in_specs=[pl.BlockSpec((R, 1), lambda j: (0, 0))],
        out_specs=pl.BlockSpec((R, BC), lambda j: (0, j)),
        out_shape=jax.ShapeDtypeStruct((R, N), jnp.float32),
        interpret=interpret,
    )(idx)
    return out
